# baseline (speedup 1.0000x reference)
import functools
import jax
import jax.numpy as jnp
from jax import lax
from jax.experimental import pallas as pl
from jax.experimental.pallas import tpu as pltpu
from jax.experimental.pallas import tpu_sc as plsc

_N = 4194304
_NC = 2
_NS = 16
_NW = _NC * _NS

_NSC = 2097152
_C = _NSC // _NW
_BLK = 8192
_NBLK = _C // _BLK
_NBUF = 4

_mesh = plsc.VectorSubcoreMesh(core_axis_name="c", subcore_axis_name="s")


@functools.partial(
    pl.kernel,
    mesh=_mesh,
    compiler_params=pltpu.CompilerParams(needs_layout_passes=False),
    out_type=jax.ShapeDtypeStruct((_NSC,), jnp.float32),
    scratch_types=[
        pltpu.VMEM((_BLK,), jnp.int32),
        pltpu.VMEM((_BLK,), jnp.int32),
        pltpu.VMEM((_BLK,), jnp.int32),
        pltpu.VMEM((_BLK,), jnp.int32),
        pltpu.VMEM((_BLK,), jnp.float32),
        pltpu.VMEM((_BLK,), jnp.float32),
        pltpu.VMEM((_BLK,), jnp.float32),
        pltpu.VMEM((_BLK,), jnp.float32),
        pltpu.VMEM((15, 1), jnp.float32),
        pltpu.VMEM((1,), jnp.float32),
        pltpu.VMEM((16,), jnp.float32),
        pltpu.SemaphoreType.DMA,
        pltpu.SemaphoreType.DMA,
        pltpu.SemaphoreType.DMA,
        pltpu.SemaphoreType.DMA,
        pltpu.SemaphoreType.DMA,
        pltpu.SemaphoreType.DMA,
        pltpu.SemaphoreType.DMA,
        pltpu.SemaphoreType.DMA,
    ],
)
def _lut_sc(x_hbm, w_hbm, b_hbm, out_hbm,
            x0, x1, x2, x3, y0, y1, y2, y3, w_v, b_v, tbl_v,
            is0, is1, is2, is3, os0, os1, os2, os3):
    pltpu.sync_copy(w_hbm, w_v)
    pltpu.sync_copy(b_hbm, b_v)
    ii = lax.iota(jnp.int32, 16)
    zeros = jnp.zeros((16,), jnp.int32)
    w16 = plsc.load_gather(w_v, [jnp.minimum(ii, 14), zeros])
    b16 = plsc.load_gather(b_v, [zeros])
    tbl_v[...] = jnp.clip(w16 + b16, 0.01, 1.0)

    wid = lax.axis_index("s") * _NC + lax.axis_index("c")
    base = wid * _C

    xb = [x0, x1, x2, x3]
    yb = [y0, y1, y2, y3]
    in_sems = [is0, is1, is2, is3]
    out_sems = [os0, os1, os2, os3]
    in_copies = [None] * _NBUF
    out_copies = [None] * _NBUF

    def start_in(i):
        s = i % _NBUF
        off = base + i * _BLK
        in_copies[s] = pltpu.async_copy(
            x_hbm.at[pl.ds(off, _BLK)], xb[s], in_sems[s])

    for i in range(_NBUF - 1):
        start_in(i)

    for i in range(_NBLK):
        s = i % _NBUF
        if i + _NBUF - 1 < _NBLK:
            start_in(i + _NBUF - 1)
        in_copies[s].wait()
        if out_copies[s] is not None:
            out_copies[s].wait()

        x_ref = xb[s]
        y_ref = yb[s]

        @plsc.parallel_loop(0, _BLK, step=16, unroll=8)
        def _(j):
            j16 = pl.multiple_of(j, 16)
            y_ref[pl.ds(j16, 16)] = plsc.load_gather(
                tbl_v, [x_ref[pl.ds(j16, 16)]])

        off = base + i * _BLK
        out_copies[s] = pltpu.async_copy(
            y_ref, out_hbm.at[pl.ds(off, _BLK)], out_sems[s])

    for c in out_copies:
        c.wait()


_LANES = 128
_ROWS = _N // _LANES
_SC_ROWS = _NSC // _LANES
_TC_ROWS = _ROWS - _SC_ROWS
_BR = 512


def _tc_body(x_ref, w_ref, b_ref, o_ref):
    xv = x_ref[...]
    b0 = b_ref[0]

    def val(k):
        return jnp.clip(w_ref[k, 0] + b0, 0.01, 1.0)

    acc = jnp.full(xv.shape, val(0), jnp.float32)
    for k in range(1, 15):
        acc = jnp.where(xv == k, val(k), acc)
    o_ref[...] = acc


def _lut_tc(x2d, W, b):
    grid = (_TC_ROWS // _BR,)
    return pl.pallas_call(
        _tc_body,
        grid=grid,
        in_specs=[
            pl.BlockSpec((_BR, _LANES), lambda i: (_SC_ROWS // _BR + i, 0)),
            pl.BlockSpec(memory_space=pltpu.SMEM),
            pl.BlockSpec(memory_space=pltpu.SMEM),
        ],
        out_specs=pl.BlockSpec((_BR, _LANES), lambda i: (_SC_ROWS // _BR + i, 0)),
        out_shape=jax.ShapeDtypeStruct((_ROWS, _LANES), jnp.float32),
    )(x2d, W, b)


def kernel(x, W, b):
    y_sc = _lut_sc(x, W, b)
    y_tc = _lut_tc(x.reshape(_ROWS, _LANES), W, b)
    y = lax.dynamic_update_slice(
        y_tc, y_sc.reshape(_SC_ROWS, _LANES), (0, 0))
    return y.reshape(_N, 1)


# speedup vs baseline: 1.3718x; 1.3718x over previous
import functools
import jax
import jax.numpy as jnp
from jax import lax
from jax.experimental import pallas as pl
from jax.experimental.pallas import tpu as pltpu
from jax.experimental.pallas import tpu_sc as plsc

_N = 4194304
_NC = 2
_NS = 16
_NW = _NC * _NS
_C = _N // _NW
_BLK = 16384
_NBLK = _C // _BLK
_NBUF = 3

_mesh = plsc.VectorSubcoreMesh(core_axis_name="c", subcore_axis_name="s")


@functools.partial(
    pl.kernel,
    mesh=_mesh,
    compiler_params=pltpu.CompilerParams(needs_layout_passes=False),
    out_type=jax.ShapeDtypeStruct((_N,), jnp.float32),
    scratch_types=(
        [pltpu.VMEM((_BLK,), jnp.int32) for _ in range(_NBUF)]
        + [pltpu.VMEM((_BLK,), jnp.float32) for _ in range(_NBUF)]
        + [pltpu.VMEM((15, 1), jnp.float32),
           pltpu.VMEM((1,), jnp.float32),
           pltpu.VMEM((16,), jnp.float32)]
        + [pltpu.SemaphoreType.DMA for _ in range(2 * _NBUF)]
    ),
)
def _lut_kernel(x_hbm, w_hbm, b_hbm, out_hbm, *scratch):
    xb = list(scratch[:_NBUF])
    yb = list(scratch[_NBUF:2 * _NBUF])
    w_v, b_v, tbl_v = scratch[2 * _NBUF:2 * _NBUF + 3]
    in_sems = list(scratch[2 * _NBUF + 3:3 * _NBUF + 3])
    out_sems = list(scratch[3 * _NBUF + 3:])

    pltpu.sync_copy(w_hbm, w_v)
    pltpu.sync_copy(b_hbm, b_v)
    ii = lax.iota(jnp.int32, 16)
    zeros = jnp.zeros((16,), jnp.int32)
    w16 = plsc.load_gather(w_v, [jnp.minimum(ii, 14), zeros])
    b16 = plsc.load_gather(b_v, [zeros])
    tbl_v[...] = jnp.clip(w16 + b16, 0.01, 1.0)

    wid = lax.axis_index("s") * _NC + lax.axis_index("c")
    base = wid * _C

    in_copies = [None] * _NBUF
    out_copies = [None] * _NBUF

    def start_in(i):
        s = i % _NBUF
        off = base + i * _BLK
        in_copies[s] = pltpu.async_copy(
            x_hbm.at[pl.ds(off, _BLK)], xb[s], in_sems[s])

    for i in range(_NBUF - 1):
        start_in(i)

    for i in range(_NBLK):
        s = i % _NBUF
        if i + _NBUF - 1 < _NBLK:
            start_in(i + _NBUF - 1)
        in_copies[s].wait()
        if out_copies[s] is not None:
            out_copies[s].wait()

        x_ref = xb[s]
        y_ref = yb[s]

        @plsc.parallel_loop(0, _BLK, step=16, unroll=8)
        def _(j):
            j16 = pl.multiple_of(j, 16)
            y_ref[pl.ds(j16, 16)] = plsc.load_gather(
                tbl_v, [x_ref[pl.ds(j16, 16)]])

        off = base + i * _BLK
        out_copies[s] = pltpu.async_copy(
            y_ref, out_hbm.at[pl.ds(off, _BLK)], out_sems[s])

    for c in out_copies:
        if c is not None:
            c.wait()


def kernel(x, W, b):
    return _lut_kernel(x, W, b).reshape(_N, 1)
